# baseline (speedup 1.0000x reference)
import functools

import jax
import jax.numpy as jnp
from jax.experimental import pallas as pl

D = 32
L = 8
K = 8

_BE = 2000
_BN = 2000
_EXP_LANES = 128


def _dot(a, b):
    return jax.lax.dot_general(
        a, b, (((1,), (0,)), ((), ())),
        precision=jax.lax.Precision.HIGHEST,
        preferred_element_type=jnp.float32,
    )


def _node_body(hacc_ref, wni_ref, wnj_ref, wsrc_ref, ni_ref, nj_ref, ns_ref):
    h = jnp.maximum(hacc_ref[...], 0.0)
    ni_ref[...] = _dot(h, wni_ref[...])
    nj_ref[...] = _dot(h, wnj_ref[...])
    ns_ref[...] = _dot(h, wsrc_ref[...])


def _node_xform(hacc, wni, wnj, wsrc):
    n = hacc.shape[0]
    grid = n // _BN
    blk = pl.BlockSpec((_BN, D), lambda i: (i, 0))
    wblk = pl.BlockSpec((D, D), lambda i: (0, 0))
    out = jax.ShapeDtypeStruct((n, D), jnp.float32)
    return pl.pallas_call(
        _node_body,
        grid=(grid,),
        in_specs=[blk, wblk, wblk, wblk],
        out_specs=[blk, blk, blk],
        out_shape=[out, out, out],
    )(hacc, wni, wnj, wsrc)


def _edge_body(ef_ref, hni_ref, hnj_ref, wf_ref, be_ref, attn_ref,
               fout_ref, logit_ref):
    x = _dot(ef_ref[...], wf_ref[...])
    x = x + hni_ref[...] + hnj_ref[...] + be_ref[...]
    f = jnp.where(x >= 0, x, 0.2 * x)
    fout_ref[...] = f
    logit_ref[...] = jnp.sum(f * attn_ref[...], axis=1, keepdims=True)


def _edge_xform(efeat, hni_s, hnj_d, wf, be, attn):
    e = efeat.shape[0]
    grid = e // _BE
    blk = pl.BlockSpec((_BE, D), lambda i: (i, 0))
    wblk = pl.BlockSpec((D, D), lambda i: (0, 0))
    vblk = pl.BlockSpec((1, D), lambda i: (0, 0))
    return pl.pallas_call(
        _edge_body,
        grid=(grid,),
        in_specs=[blk, blk, blk, wblk, vblk, vblk],
        out_specs=[blk, pl.BlockSpec((_BE, 1), lambda i: (i, 0))],
        out_shape=[jax.ShapeDtypeStruct((e, D), jnp.float32),
                   jax.ShapeDtypeStruct((e, 1), jnp.float32)],
    )(efeat, hni_s, hnj_d, wf, be.reshape(1, D), attn.reshape(1, D))


def _exp_body(lg_ref, m_ref, ex_ref):
    ex_ref[...] = jnp.exp(lg_ref[...] - m_ref[...])


def _exp_kernel(logits, m_dst):
    e = logits.shape[0]
    rows = e // _EXP_LANES
    brows = 512
    blk = pl.BlockSpec((brows, _EXP_LANES), lambda i: (i, 0))
    lg2 = logits.reshape(rows, _EXP_LANES)
    m2 = m_dst.reshape(rows, _EXP_LANES)
    out = pl.pallas_call(
        _exp_body,
        grid=(pl.cdiv(rows, brows),),
        in_specs=[blk, blk],
        out_specs=blk,
        out_shape=jax.ShapeDtypeStruct((rows, _EXP_LANES), jnp.float32),
    )(lg2, m2)
    return out.reshape(e)


def _msg_body(ex_ref, s_ref, hs_ref, msg_ref):
    alpha = ex_ref[...] / (s_ref[...] + 1e-9)
    msg_ref[...] = alpha * hs_ref[...]


def _msg_kernel(ex, s_dst, hsrc_s):
    e = ex.shape[0]
    grid = e // _BE
    blk = pl.BlockSpec((_BE, D), lambda i: (i, 0))
    sblk = pl.BlockSpec((_BE, 1), lambda i: (i, 0))
    return pl.pallas_call(
        _msg_body,
        grid=(grid,),
        in_specs=[sblk, sblk, blk],
        out_specs=blk,
        out_shape=jax.ShapeDtypeStruct((e, D), jnp.float32),
    )(ex.reshape(e, 1), s_dst.reshape(e, 1), hsrc_s)


def _head_body(p_ref, wl_ref, bl_ref, w1_ref, b1_ref, w2_ref, b2_ref,
               wc_ref, bc_ref, out_ref):
    x = jnp.maximum(_dot(p_ref[...], wl_ref[...]) + bl_ref[...], 0.0)
    x = jnp.maximum(_dot(x, w1_ref[...]) + b1_ref[...], 0.0)
    x = jnp.maximum(_dot(x, w2_ref[...]) + b2_ref[...], 0.0)
    out_ref[...] = _dot(x, wc_ref[...]) + bc_ref[...]


def _head(pooled, wl, bl, w1, b1, w2, b2, wc, bc):
    return pl.pallas_call(
        _head_body,
        out_shape=jax.ShapeDtypeStruct((1, 2), jnp.float32),
    )(pooled, wl, bl.reshape(1, D), w1, b1.reshape(1, D),
      w2, b2.reshape(1, D), wc, bc.reshape(1, 2))


def kernel(h, e, edge_index, node_emb, edge_emb, W_ni, W_fij, W_nj, W_src,
           attn, bias_e, W_lin, b_lin, W_lin1, b_lin1, W_lin2, b_lin2,
           W_cls, b_cls):
    src = edge_index[0]
    dst = edge_index[1]
    n = h.shape[0]

    hacc = jnp.take(node_emb, h, axis=0)
    efeat = jnp.take(edge_emb, e, axis=0)

    for i in range(L):
        ni, nj, ns = _node_xform(hacc, W_ni[i], W_nj[i], W_src[i])
        hni_s = jnp.take(ni, src, axis=0)
        hnj_d = jnp.take(nj, dst, axis=0)
        f_out, logits2 = _edge_xform(efeat, hni_s, hnj_d, W_fij[i],
                                     bias_e[i], attn[i])
        logits = logits2.reshape(-1)
        m = jax.ops.segment_max(logits, dst, num_segments=n)
        m = jnp.where(jnp.isfinite(m), m, 0.0)
        ex = _exp_kernel(logits, jnp.take(m, dst))
        s = jax.ops.segment_sum(ex, dst, num_segments=n)
        hsrc_s = jnp.take(ns, src, axis=0)
        msg = _msg_kernel(ex, jnp.take(s, dst), hsrc_s)
        hacc = jax.ops.segment_sum(msg, dst, num_segments=n)
        efeat = f_out

    hfeat = jnp.maximum(hacc, 0.0)
    hs = jnp.sort(hfeat, axis=-1)
    _, idx = jax.lax.top_k(hs[:, -1], K)
    pooled = hs[idx].reshape(1, K * D)
    return _head(pooled, W_lin, b_lin, W_lin1, b_lin1, W_lin2, b_lin2,
                 W_cls, b_cls)
